# baseline (device time: 14943 ns/iter reference)
import jax
import jax.numpy as jnp
from jax import lax
from jax.experimental import pallas as pl
from jax.experimental.pallas import tpu as pltpu

N_CHUNKS = 4


def kernel(ids, E):
    t = ids.shape[0]
    v_local, d = E.shape
    tc = t // N_CHUNKS

    def body(ids_ref, E_ref, out_ref, eq_ref, send_ref, recv_ref,
             sscale_ref, rscale_ref, send_sems, recv_sems,
             scale_send_sem, scale_recv_sem):
        my_x = lax.axis_index("x")
        my_y = lax.axis_index("y")
        my_z = lax.axis_index("z")
        nbr = (my_x, my_y, 1 - my_z)

        barrier_sem = pltpu.get_barrier_semaphore()
        pl.semaphore_signal(
            barrier_sem, inc=1, device_id=nbr,
            device_id_type=pl.DeviceIdType.MESH,
        )

        s = jnp.max(jnp.abs(E_ref[...])) * (1.02 / 127.0)
        inv_s = 1.0 / s
        eq_ref[...] = jnp.round(E_ref[...] * inv_s).astype(jnp.int8)
        sscale_ref[...] = jnp.full((8, 128), s, jnp.float32)

        pl.semaphore_wait(barrier_sem, 1)
        scale_rdma = pltpu.make_async_remote_copy(
            src_ref=sscale_ref,
            dst_ref=rscale_ref,
            send_sem=scale_send_sem,
            recv_sem=scale_recv_sem,
            device_id=nbr,
            device_id_type=pl.DeviceIdType.MESH,
        )
        scale_rdma.start()

        base = my_z * v_local
        cols16 = lax.broadcasted_iota(jnp.int16, (tc, v_local), 1)
        rdmas = []
        for c in range(N_CHUNKS):
            rows = pl.ds(c * tc, tc)
            idsc = (ids_ref[rows, :] - base).astype(jnp.int16)
            onehot = (cols16 == idsc).astype(jnp.int8)
            q = jnp.dot(
                onehot, eq_ref[...],
                preferred_element_type=jnp.int32,
            )
            send_ref[rows, :] = q.astype(jnp.int8)
            rdma = pltpu.make_async_remote_copy(
                src_ref=send_ref.at[rows, :],
                dst_ref=recv_ref.at[rows, :],
                send_sem=send_sems.at[c],
                recv_sem=recv_sems.at[c],
                device_id=nbr,
                device_id_type=pl.DeviceIdType.MESH,
            )
            rdma.start()
            rdmas.append(rdma)

        scale_rdma.wait_recv()
        s_peer = rscale_ref[0, 0]
        for c in range(N_CHUNKS):
            rows = pl.ds(c * tc, tc)
            rdmas[c].wait_recv()
            out_ref[rows, :] = (
                send_ref[rows, :].astype(jnp.float32) * s
                + recv_ref[rows, :].astype(jnp.float32) * s_peer
            )

        scale_rdma.wait_send()
        for c in range(N_CHUNKS):
            rdmas[c].wait_send()

    return pl.pallas_call(
        body,
        out_shape=jax.ShapeDtypeStruct((t, d), jnp.float32),
        in_specs=[
            pl.BlockSpec(memory_space=pltpu.VMEM),
            pl.BlockSpec(memory_space=pltpu.VMEM),
        ],
        out_specs=pl.BlockSpec(memory_space=pltpu.VMEM),
        scratch_shapes=[
            pltpu.VMEM((v_local, d), jnp.int8),
            pltpu.VMEM((t, d), jnp.int8),
            pltpu.VMEM((t, d), jnp.int8),
            pltpu.VMEM((8, 128), jnp.float32),
            pltpu.VMEM((8, 128), jnp.float32),
            pltpu.SemaphoreType.DMA((N_CHUNKS,)),
            pltpu.SemaphoreType.DMA((N_CHUNKS,)),
            pltpu.SemaphoreType.DMA,
            pltpu.SemaphoreType.DMA,
        ],
        compiler_params=pltpu.CompilerParams(collective_id=0),
    )(ids.reshape(t, 1), E)
